# Initial kernel scaffold; baseline (speedup 1.0000x reference)
import jax
import jax.numpy as jnp
from jax.experimental import pallas as pl


def kernel(x, W_pe, b_pe, W_s1, b_s1, W_ss, W_sr, b_sr, W_f1, b_f1, W_f2, b_f2, W_k1, b_k1, W_k2, b_k2, W_m1, b_m1, W_m2, b_m2, W_l1, b_l1, W_l2, b_l2, W_g1, b_g1, W_g2, b_g2):
    raise NotImplementedError("write your pallas kernel here")


# speedup vs baseline: 13.8350x; 13.8350x over previous
import functools

import jax
import jax.numpy as jnp
from jax import lax
from jax.experimental import pallas as pl
from jax.experimental.pallas import tpu as pltpu
import jax.experimental.pallas.tpu_sc as plsc

B, N, M, K = 2, 10000, 512, 16
PE, H, CF = 128, 64, 256
NBLK = 2000
NB = N // NBLK
MBLK = 128
MB = M // MBLK

_NEG = -1e30
_INF = 3e38


def _ln(x):
    mu = jnp.mean(x, axis=-1, keepdims=True)
    v = jnp.mean((x - mu) * (x - mu), axis=-1, keepdims=True)
    return (x - mu) / jnp.sqrt(v + 1e-5)


def _dotT(a, b):
    return lax.dot_general(a, b, (((0,), (0,)), ((), ())),
                           preferred_element_type=jnp.float32)


def _stream_body(x_ref, Ws1_ref, bs1_ref, Wss_ref, Wf1_ref, bf1_ref,
                 Wf2_ref, bf2_ref, Wpe_ref, bpe_ref,
                 table_ref, skt_ref, skft_ref,
                 mx_ref, s_ref, accp_ref, accf_ref):
    nb = pl.program_id(1)

    @pl.when(nb == 0)
    def _init():
        mx_ref[...] = jnp.full((1, M), _NEG, jnp.float32)
        s_ref[...] = jnp.zeros((1, M), jnp.float32)
        accp_ref[...] = jnp.zeros((3, M), jnp.float32)
        accf_ref[...] = jnp.zeros((H, M), jnp.float32)

    xb = x_ref[0]
    feat = jax.nn.relu(
        jnp.dot(xb, Ws1_ref[...], preferred_element_type=jnp.float32)
        + bs1_ref[...])
    z = jnp.dot(feat, Wss_ref[...],
                preferred_element_type=jnp.float32)

    mx_old = mx_ref[...]
    mx_new = jnp.maximum(mx_old, jnp.max(z, axis=0, keepdims=True))
    scale = jnp.exp(mx_old - mx_new)
    e = jnp.exp(z - mx_new)
    s_new = s_ref[...] * scale + jnp.sum(e, axis=0, keepdims=True)
    accp_new = accp_ref[...] * scale + _dotT(xb, e)
    accf_new = accf_ref[...] * scale + _dotT(feat, e)
    mx_ref[...] = mx_new
    s_ref[...] = s_new
    accp_ref[...] = accp_new
    accf_ref[...] = accf_new

    h1 = jax.nn.relu(
        jnp.dot(xb, Wf1_ref[...], preferred_element_type=jnp.float32)
        + bf1_ref[...])
    sff = jnp.dot(h1, Wf2_ref[...],
                  preferred_element_type=jnp.float32) + bf2_ref[...]
    pe = jnp.dot(xb, Wpe_ref[...],
                 preferred_element_type=jnp.float32) + bpe_ref[...]
    table_ref[0, :, 0:PE] = pe
    table_ref[0, :, PE:PE + CF] = sff

    @pl.when(nb == NB - 1)
    def _fin():
        skt_ref[0] = accp_new / s_new
        skft_ref[0] = accf_new / s_new


def _run_stream(x, W_s1, b_s1, W_ss, W_f1, b_f1, W_f2, b_f2, W_pe, b_pe):
    wspec = lambda shp: pl.BlockSpec(shp, lambda b, nb: (0,) * len(shp))
    return pl.pallas_call(
        _stream_body,
        grid=(B, NB),
        in_specs=[
            pl.BlockSpec((1, NBLK, 3), lambda b, nb: (b, nb, 0)),
            wspec((3, H)), wspec((H,)), wspec((H, M)),
            wspec((3, PE)), wspec((PE,)), wspec((PE, CF)), wspec((CF,)),
            wspec((3, PE)), wspec((PE,)),
        ],
        out_specs=[
            pl.BlockSpec((1, NBLK, PE + CF), lambda b, nb: (b, nb, 0)),
            pl.BlockSpec((1, 3, M), lambda b, nb: (b, 0, 0)),
            pl.BlockSpec((1, H, M), lambda b, nb: (b, 0, 0)),
        ],
        out_shape=[
            jax.ShapeDtypeStruct((B, N, PE + CF), jnp.float32),
            jax.ShapeDtypeStruct((B, 3, M), jnp.float32),
            jax.ShapeDtypeStruct((B, H, M), jnp.float32),
        ],
        scratch_shapes=[
            pltpu.VMEM((1, M), jnp.float32),
            pltpu.VMEM((1, M), jnp.float32),
            pltpu.VMEM((3, M), jnp.float32),
            pltpu.VMEM((H, M), jnp.float32),
        ],
    )(x, W_s1, b_s1, W_ss, W_f1, b_f1, W_f2, b_f2, W_pe, b_pe)


def _heads_body(skt_ref, skft_ref, Wsr_ref, bsr_ref, Wk1_ref, bk1_ref,
                Wk2_ref, bk2_ref, Wpe_ref, bpe_ref,
                rt_ref, skemb_ref, skf0_ref):
    skt = skt_ref[0]
    skft = skft_ref[0]
    rz = _dotT(Wsr_ref[...], skft) + bsr_ref[...][0]
    rt = jnp.maximum(rz, 0.0) + jnp.log1p(jnp.exp(-jnp.abs(rz)))
    rt_ref[0] = rt
    skemb_ref[0] = _dotT(skt, Wpe_ref[...]) + bpe_ref[...]
    cat = jnp.concatenate([skt, rt], axis=0)
    h = jax.nn.relu(_dotT(cat, Wk1_ref[...]) + bk1_ref[...])
    skf0_ref[0] = jnp.dot(h, Wk2_ref[...],
                          preferred_element_type=jnp.float32) + bk2_ref[...]


def _run_heads(skt, skft, W_sr, b_sr, W_k1, b_k1, W_k2, b_k2, W_pe, b_pe):
    wspec = lambda shp: pl.BlockSpec(shp, lambda b: (0,) * len(shp))
    return pl.pallas_call(
        _heads_body,
        grid=(B,),
        in_specs=[
            pl.BlockSpec((1, 3, M), lambda b: (b, 0, 0)),
            pl.BlockSpec((1, H, M), lambda b: (b, 0, 0)),
            wspec((H, 1)), wspec((1,)),
            wspec((4, PE)), wspec((PE,)), wspec((PE, CF)), wspec((CF,)),
            wspec((3, PE)), wspec((PE,)),
        ],
        out_specs=[
            pl.BlockSpec((1, 1, M), lambda b: (b, 0, 0)),
            pl.BlockSpec((1, M, PE), lambda b: (b, 0, 0)),
            pl.BlockSpec((1, M, CF), lambda b: (b, 0, 0)),
        ],
        out_shape=[
            jax.ShapeDtypeStruct((B, 1, M), jnp.float32),
            jax.ShapeDtypeStruct((B, M, PE), jnp.float32),
            jax.ShapeDtypeStruct((B, M, CF), jnp.float32),
        ],
    )(skt, skft, W_sr, b_sr, W_k1, b_k1, W_k2, b_k2, W_pe, b_pe)


def _topk_body(sk_ref, xt_ref, idx_ref):
    sk = sk_ref[0]
    xt = xt_ref[0]
    d2 = ((sk[:, 0:1] - xt[0:1, :]) ** 2
          + (sk[:, 1:2] - xt[1:2, :]) ** 2) \
        + (sk[:, 2:3] - xt[2:3, :]) ** 2
    iota = lax.broadcasted_iota(jnp.int32, (MBLK, N), 1)
    cols = []
    for _ in range(K):
        vmin = jnp.min(d2, axis=1, keepdims=True)
        cand = jnp.where(d2 == vmin, iota, N)
        ik = jnp.min(cand, axis=1, keepdims=True)
        cols.append(ik)
        d2 = jnp.where(iota == ik, _INF, d2)
    idx_ref[0] = jnp.concatenate(cols, axis=1)


def _run_topk(sk, x_t):
    return pl.pallas_call(
        _topk_body,
        grid=(B, MB),
        in_specs=[
            pl.BlockSpec((1, MBLK, 3), lambda b, mb: (b, mb, 0)),
            pl.BlockSpec((1, 3, N), lambda b, mb: (b, 0, 0)),
        ],
        out_specs=pl.BlockSpec((1, MBLK, K), lambda b, mb: (b, mb, 0)),
        out_shape=jax.ShapeDtypeStruct((B, M, K), jnp.int32),
    )(sk, x_t)


ROWS = B * M * K
NW = 32
RPW = ROWS // NW
CH = 128
NCH = RPW // CH


def _gather_sc(idx_flat, table_flat):
    mesh = plsc.VectorSubcoreMesh(core_axis_name="c", subcore_axis_name="s")

    @functools.partial(
        pl.kernel,
        out_type=jax.ShapeDtypeStruct((ROWS, PE + CF), jnp.float32),
        mesh=mesh,
        scratch_types=[
            pltpu.VMEM((RPW,), jnp.int32),
            pltpu.VMEM((CH, PE + CF), jnp.float32),
            pltpu.SemaphoreType.DMA,
        ],
    )
    def k(idx_hbm, table_hbm, out_hbm, idx_v, rows_v, sem):
        wid = lax.axis_index("s") * 2 + lax.axis_index("c")
        base = wid * RPW
        bofs = (base // (M * K)) * N
        pltpu.sync_copy(idx_hbm.at[pl.ds(base, RPW)], idx_v)
        for i in range(RPW // 16):
            sl = pl.ds(i * 16, 16)
            idx_v[sl] = idx_v[sl] + bofs
        for c in range(NCH):
            pltpu.async_copy(
                table_hbm.at[idx_v.at[pl.ds(c * CH, CH)]], rows_v, sem
            ).wait()
            pltpu.sync_copy(rows_v, out_hbm.at[pl.ds(base + c * CH, CH)])

    return k(idx_flat, table_flat)


def _group_body(grp_ref, skemb_ref, skf0_ref,
                Wm1_ref, bm1_ref, Wm2_ref, bm2_ref,
                Wl1_ref, bl1_ref, Wl2_ref, bl2_ref,
                Wg1_ref, bg1_ref, Wg2_ref, bg2_ref,
                lf_ref, gf_ref, gmax_ref, gsum_ref):
    mb = pl.program_id(1)

    @pl.when(mb == 0)
    def _init():
        gmax_ref[...] = jnp.full((1, 2 * CF), _NEG, jnp.float32)
        gsum_ref[...] = jnp.zeros((1, 2 * CF), jnp.float32)

    grp = grp_ref[0]
    skemb = skemb_ref[0]
    skf0 = skf0_ref[0]
    Wm1 = Wm1_ref[...]
    bm1 = bm1_ref[...]

    base = jnp.dot(grp, Wm1, preferred_element_type=jnp.float32) + bm1
    corr = jnp.dot(skemb, Wm1[0:PE, :],
                   preferred_element_type=jnp.float32)
    h1n = base.reshape(MBLK, K, CF) - corr.reshape(MBLK, 1, CF)
    h1n = jax.nn.relu(_ln(h1n)).reshape(MBLK * K, CF)
    h1c = jax.nn.relu(_ln(
        jnp.dot(skf0, Wm1[PE:PE + CF, :],
                preferred_element_type=jnp.float32) + bm1))

    gn = jax.nn.relu(_ln(
        jnp.dot(h1n, Wm2_ref[...], preferred_element_type=jnp.float32)
        + bm2_ref[...]))
    gc = jax.nn.relu(_ln(
        jnp.dot(h1c, Wm2_ref[...], preferred_element_type=jnp.float32)
        + bm2_ref[...]))

    g3 = gn.reshape(MBLK, K, CF)
    gmx = jnp.maximum(jnp.max(g3, axis=1), gc)
    gmean = (jnp.sum(g3, axis=1) + gc) * (1.0 / (K + 1))
    skf = jnp.concatenate([gmx, gmean], axis=1)

    lf = jax.nn.relu(_ln(
        jnp.dot(skf, Wl1_ref[...], preferred_element_type=jnp.float32)
        + bl1_ref[...]))
    lf = jax.nn.relu(_ln(
        jnp.dot(lf, Wl2_ref[...], preferred_element_type=jnp.float32)
        + bl2_ref[...]))
    lf_ref[0] = lf

    gmax_ref[...] = jnp.maximum(gmax_ref[...],
                                jnp.max(skf, axis=0, keepdims=True))
    gsum_ref[...] = gsum_ref[...] + jnp.sum(skf, axis=0, keepdims=True)

    @pl.when(mb == MB - 1)
    def _fin():
        gskf = jnp.concatenate(
            [gmax_ref[...], gsum_ref[...] * (1.0 / M)], axis=1)
        gf = jax.nn.relu(_ln(
            jnp.dot(gskf, Wg1_ref[...], preferred_element_type=jnp.float32)
            + bg1_ref[...]))
        gf = jax.nn.relu(_ln(
            jnp.dot(gf, Wg2_ref[...], preferred_element_type=jnp.float32)
            + bg2_ref[...]))
        gf_ref[0] = gf


def _run_group(grp, skemb, skf0, W_m1, b_m1, W_m2, b_m2,
               W_l1, b_l1, W_l2, b_l2, W_g1, b_g1, W_g2, b_g2):
    wspec = lambda shp: pl.BlockSpec(shp, lambda b, mb: (0,) * len(shp))
    return pl.pallas_call(
        _group_body,
        grid=(B, MB),
        in_specs=[
            pl.BlockSpec((1, MBLK * K, PE + CF), lambda b, mb: (b, mb, 0)),
            pl.BlockSpec((1, MBLK, PE), lambda b, mb: (b, mb, 0)),
            pl.BlockSpec((1, MBLK, CF), lambda b, mb: (b, mb, 0)),
            wspec((PE + CF, CF)), wspec((CF,)),
            wspec((CF, CF)), wspec((CF,)),
            wspec((2 * CF, 2 * CF)), wspec((2 * CF,)),
            wspec((2 * CF, 2 * CF)), wspec((2 * CF,)),
            wspec((4 * CF, 2 * CF)), wspec((2 * CF,)),
            wspec((2 * CF, 2 * CF)), wspec((2 * CF,)),
        ],
        out_specs=[
            pl.BlockSpec((1, MBLK, 2 * CF), lambda b, mb: (b, mb, 0)),
            pl.BlockSpec((1, 1, 2 * CF), lambda b, mb: (b, 0, 0)),
        ],
        out_shape=[
            jax.ShapeDtypeStruct((B, M, 2 * CF), jnp.float32),
            jax.ShapeDtypeStruct((B, 1, 2 * CF), jnp.float32),
        ],
        scratch_shapes=[
            pltpu.VMEM((1, 2 * CF), jnp.float32),
            pltpu.VMEM((1, 2 * CF), jnp.float32),
        ],
    )(grp, skemb, skf0, W_m1, b_m1, W_m2, b_m2,
      W_l1, b_l1, W_l2, b_l2, W_g1, b_g1, W_g2, b_g2)


def kernel(x, W_pe, b_pe, W_s1, b_s1, W_ss, W_sr, b_sr, W_f1, b_f1,
           W_f2, b_f2, W_k1, b_k1, W_k2, b_k2, W_m1, b_m1, W_m2, b_m2,
           W_l1, b_l1, W_l2, b_l2, W_g1, b_g1, W_g2, b_g2):
    table, skt, skft = _run_stream(
        x, W_s1, b_s1, W_ss, W_f1, b_f1, W_f2, b_f2, W_pe, b_pe)
    rt, skemb, skf0 = _run_heads(
        skt, skft, W_sr, b_sr, W_k1, b_k1, W_k2, b_k2, W_pe, b_pe)
    sk = skt.transpose(0, 2, 1)
    r = rt.transpose(0, 2, 1)
    idx = _run_topk(sk, x.transpose(0, 2, 1))
    grouped = _gather_sc(idx.reshape(ROWS),
                         table.reshape(B * N, PE + CF))
    lf, gf = _run_group(
        grouped.reshape(B, M * K, PE + CF), skemb, skf0,
        W_m1, b_m1, W_m2, b_m2, W_l1, b_l1, W_l2, b_l2,
        W_g1, b_g1, W_g2, b_g2)
    return (lf, gf, sk, r)
